# baseline (speedup 1.0000x reference)
import functools

import jax
import jax.numpy as jnp
from jax import lax
from jax.experimental import pallas as pl
from jax.experimental.pallas import tpu as pltpu
from jax.experimental.pallas import tpu_sc as plsc

VOCAB = 100000
EMBED = 128
CTX = 4096
B = 4
T = 4096

_info = plsc.get_sparse_core_info()
NC, NS, L = _info.num_cores, _info.num_subcores, _info.num_lanes
NW = NC * NS
TBLK = T // NW
HALF = 2
CROWS = TBLK // HALF
NCH = B * HALF
NBUF = 3


def _body(x_hbm, tok_hbm, pos_hbm, out_hbm,
          idx_v, tok_v, pos_v,
          sem_p, sem_i0, sem_i1, sem_i2, sem_i3,
          sem_i4, sem_i5, sem_i6, sem_i7,
          sem_g0, sem_g1, sem_g2, sem_o0, sem_o1, sem_o2):
    sems_i = (sem_i0, sem_i1, sem_i2, sem_i3,
              sem_i4, sem_i5, sem_i6, sem_i7)
    sems_g = (sem_g0, sem_g1, sem_g2)
    sems_o = (sem_o0, sem_o1, sem_o2)
    wid = lax.axis_index("s") * NC + lax.axis_index("c")
    t0 = wid * TBLK

    def bh(c):
        return c // HALF, (c % HALF) * CROWS

    p_desc = pltpu.async_copy(pos_hbm.at[pl.ds(t0, TBLK)], pos_v, sem_p)
    i_descs = []
    for c in range(NCH):
        b, off = bh(c)
        i_descs.append(pltpu.async_copy(
            x_hbm.at[b, pl.ds(t0 + off, CROWS)], idx_v.at[c], sems_i[c]))

    def start_gather(c):
        return pltpu.async_copy(tok_hbm.at[idx_v.at[c]],
                                tok_v.at[c % NBUF], sems_g[c % NBUF])

    g = [None] * NCH
    o = [None] * NCH
    for c in range(NBUF - 1):
        i_descs[c].wait()
        g[c] = start_gather(c)
    for c in range(NCH):
        buf = c % NBUF
        n = c + NBUF - 1
        if n < NCH:
            if n >= NBUF:
                o[n - NBUF].wait()
            i_descs[n].wait()
            g[n] = start_gather(n)
        g[c].wait()
        if c == 0:
            p_desc.wait()

        b, off = bh(c)

        @plsc.parallel_loop(0, CROWS, unroll=2)
        def add_row(r):
            for j in range(EMBED // L):
                d = pl.ds(j * L, L)
                plsc.addupdate(tok_v.at[buf, r, d], pos_v[off + r, d])

        o[c] = pltpu.async_copy(tok_v.at[buf],
                                out_hbm.at[b, pl.ds(t0 + off, CROWS)],
                                sems_o[buf])
    for c in range(NCH - NBUF, NCH):
        o[c].wait()


_mesh = plsc.VectorSubcoreMesh(core_axis_name="c", subcore_axis_name="s")

_sc_call = functools.partial(
    pl.kernel,
    out_type=jax.ShapeDtypeStruct((B, T, EMBED), jnp.float32),
    mesh=_mesh,
    scratch_types=[
        pltpu.VMEM((NCH, CROWS), jnp.int32),
        pltpu.VMEM((NBUF, CROWS, EMBED), jnp.float32),
        pltpu.VMEM((TBLK, EMBED), jnp.float32),
    ] + [pltpu.SemaphoreType.DMA] * 15,
)(_body)


def kernel(x, tok_table, pos_table):
    return _sc_call(x.astype(jnp.int32), tok_table, pos_table)
